# baseline (speedup 1.0000x reference)
import jax
import jax.numpy as jnp
from jax import lax
from jax.experimental import pallas as pl
from jax.experimental.pallas import tpu as pltpu
from jax.experimental.pallas import tpu_sc as plsc

_NC = 2
_NS = 16
_NW = _NC * _NS
_B = 16384
_D = 64
_BPW = _B // _NW
_BLK = 16
_NBLK = _BPW // _BLK
_DEPTH = 8


def _gather_body(utab, itab, uidx, iidx, ue_out, ie_out,
                 uidx_v, iidx_v, usem, isem):
    wid = lax.axis_index("s") * _NC + lax.axis_index("c")
    base = wid * _BPW
    pltpu.sync_copy(uidx.at[wid], uidx_v)
    pltpu.sync_copy(iidx.at[wid], iidx_v)

    def step(g, _):
        uvec = uidx_v[pl.ds(g * _BLK, _BLK)]
        ivec = iidx_v[pl.ds(g * _BLK, _BLK)]
        for k in range(_BLK):
            pltpu.async_copy(utab.at[pl.ds(uvec[k], 1)],
                             ue_out.at[pl.ds(base + g * _BLK + k, 1)], usem)
            pltpu.async_copy(itab.at[pl.ds(ivec[k], 1)],
                             ie_out.at[pl.ds(base + g * _BLK + k, 1)], isem)
        return ()

    lax.fori_loop(0, _NBLK, step, ())

    pltpu.make_async_copy(
        utab.at[pl.ds(0, _BPW)], ue_out.at[pl.ds(base, _BPW)], usem).wait()
    pltpu.make_async_copy(
        itab.at[pl.ds(0, _BPW)], ie_out.at[pl.ds(base, _BPW)], isem).wait()


def _make_gather():
    return pl.kernel(
        _gather_body,
        out_type=(jax.ShapeDtypeStruct((_B, _D), jnp.float32),
                  jax.ShapeDtypeStruct((_B, _D), jnp.float32)),
        mesh=plsc.VectorSubcoreMesh(core_axis_name="c", subcore_axis_name="s",
                                    num_cores=_NC, num_subcores=_NS),
        scratch_types=[
            pltpu.VMEM((_BPW,), jnp.int32),
            pltpu.VMEM((_BPW,), jnp.int32),
            pltpu.SemaphoreType.DMA,
            pltpu.SemaphoreType.DMA,
        ],
    )


def _mlp_body(ue, ie, w1u, w1i, b1, g1, be1,
              w2, b2, g2, be2, w3, b3, out):
    h = jnp.dot(ue[...], w1u[...], preferred_element_type=jnp.float32)
    h = h + jnp.dot(ie[...], w1i[...], preferred_element_type=jnp.float32)
    h = h + b1[...]
    m = jnp.mean(h, axis=0, keepdims=True)
    v = jnp.mean(jnp.square(h - m), axis=0, keepdims=True)
    h = (h - m) * lax.rsqrt(v + 1e-5) * g1[...] + be1[...]
    h = jnp.maximum(h, 0.0)
    h2 = jnp.dot(h, w2[...], preferred_element_type=jnp.float32) + b2[...]
    m2 = jnp.mean(h2, axis=0, keepdims=True)
    v2 = jnp.mean(jnp.square(h2 - m2), axis=0, keepdims=True)
    h2 = (h2 - m2) * lax.rsqrt(v2 + 1e-5) * g2[...] + be2[...]
    h2 = jnp.maximum(h2, 0.0)
    z = jnp.sum(h2 * w3[...], axis=1) + b3[0, 0]
    out[...] = jax.nn.sigmoid(z)


def _mlp(*args):
    return pl.pallas_call(
        _mlp_body,
        out_shape=jax.ShapeDtypeStruct((_B,), jnp.float32),
        compiler_params=pltpu.CompilerParams(vmem_limit_bytes=100 * 1024 * 1024),
    )(*args)


def kernel(users, items, user_table, item_table,
           W1, b1, g1, be1, W2, b2, g2, be2, W3, b3):
    uidx = users.reshape(_NW, _BPW)
    iidx = items.reshape(_NW, _BPW)
    ue, ie = _make_gather()(user_table, item_table, uidx, iidx)
    w1u = W1[:, :_D].T
    w1i = W1[:, _D:].T
    return _mlp(ue, ie, w1u, w1i,
                b1.reshape(1, -1), g1.reshape(1, -1), be1.reshape(1, -1),
                W2.T, b2.reshape(1, -1), g2.reshape(1, -1), be2.reshape(1, -1),
                W3, b3.reshape(1, 1))


# speedup vs baseline: 1.6719x; 1.6719x over previous
import jax
import jax.numpy as jnp
from jax import lax
from jax.experimental import pallas as pl
from jax.experimental.pallas import tpu as pltpu
from jax.experimental.pallas import tpu_sc as plsc

_NC = 2
_NS = 16
_NW = _NC * _NS
_B = 16384
_D = 64
_BPW = _B // _NW
_BLK = 16
_NBLK = _BPW // _BLK
_DEPTH = 8


def _gather_body(utab, itab, uidx, iidx, ue_out, ie_out,
                 uidx_v, iidx_v, urows_v, irows_v, usem, isem):
    wid = lax.axis_index("s") * _NC + lax.axis_index("c")
    base = wid * _BPW
    pltpu.sync_copy(uidx.at[wid], uidx_v)
    pltpu.sync_copy(iidx.at[wid], iidx_v)

    half = _BPW // 2
    for p in range(2):
        off = p * half

        def step(g, _):
            uvec = uidx_v[pl.ds(off + g * _BLK, _BLK)]
            ivec = iidx_v[pl.ds(off + g * _BLK, _BLK)]
            for k in range(_BLK):
                pltpu.async_copy(utab.at[pl.ds(uvec[k], 1)],
                                 urows_v.at[pl.ds(g * _BLK + k, 1)], usem)
                pltpu.async_copy(itab.at[pl.ds(ivec[k], 1)],
                                 irows_v.at[pl.ds(g * _BLK + k, 1)], isem)
            return ()

        lax.fori_loop(0, _NBLK // 2, step, ())

        pltpu.make_async_copy(utab.at[pl.ds(0, half)], urows_v, usem).wait()
        pltpu.make_async_copy(itab.at[pl.ds(0, half)], irows_v, isem).wait()
        pltpu.sync_copy(urows_v, ue_out.at[pl.ds(base + off, half)])
        pltpu.sync_copy(irows_v, ie_out.at[pl.ds(base + off, half)])


def _make_gather():
    return pl.kernel(
        _gather_body,
        out_type=(jax.ShapeDtypeStruct((_B, _D), jnp.float32),
                  jax.ShapeDtypeStruct((_B, _D), jnp.float32)),
        mesh=plsc.VectorSubcoreMesh(core_axis_name="c", subcore_axis_name="s",
                                    num_cores=_NC, num_subcores=_NS),
        scratch_types=[
            pltpu.VMEM((_BPW,), jnp.int32),
            pltpu.VMEM((_BPW,), jnp.int32),
            pltpu.VMEM((_BPW // 2, _D), jnp.float32),
            pltpu.VMEM((_BPW // 2, _D), jnp.float32),
            pltpu.SemaphoreType.DMA,
            pltpu.SemaphoreType.DMA,
        ],
    )


def _mlp_body(ue, ie, w1u, w1i, b1, g1, be1,
              w2, b2, g2, be2, w3, b3, out):
    h = jnp.dot(ue[...], w1u[...], preferred_element_type=jnp.float32)
    h = h + jnp.dot(ie[...], w1i[...], preferred_element_type=jnp.float32)
    h = h + b1[...]
    m = jnp.mean(h, axis=0, keepdims=True)
    v = jnp.mean(jnp.square(h - m), axis=0, keepdims=True)
    h = (h - m) * lax.rsqrt(v + 1e-5) * g1[...] + be1[...]
    h = jnp.maximum(h, 0.0)
    h2 = jnp.dot(h, w2[...], preferred_element_type=jnp.float32) + b2[...]
    m2 = jnp.mean(h2, axis=0, keepdims=True)
    v2 = jnp.mean(jnp.square(h2 - m2), axis=0, keepdims=True)
    h2 = (h2 - m2) * lax.rsqrt(v2 + 1e-5) * g2[...] + be2[...]
    h2 = jnp.maximum(h2, 0.0)
    z = jnp.sum(h2 * w3[...], axis=1) + b3[0, 0]
    out[...] = jax.nn.sigmoid(z)


def _mlp(*args):
    return pl.pallas_call(
        _mlp_body,
        out_shape=jax.ShapeDtypeStruct((_B,), jnp.float32),
        compiler_params=pltpu.CompilerParams(vmem_limit_bytes=100 * 1024 * 1024),
    )(*args)


def kernel(users, items, user_table, item_table,
           W1, b1, g1, be1, W2, b2, g2, be2, W3, b3):
    uidx = users.reshape(_NW, _BPW)
    iidx = items.reshape(_NW, _BPW)
    ue, ie = _make_gather()(user_table, item_table, uidx, iidx)
    w1u = W1[:, :_D].T
    w1i = W1[:, _D:].T
    return _mlp(ue, ie, w1u, w1i,
                b1.reshape(1, -1), g1.reshape(1, -1), be1.reshape(1, -1),
                W2.T, b2.reshape(1, -1), g2.reshape(1, -1), be2.reshape(1, -1),
                W3, b3.reshape(1, 1))
